# Initial kernel scaffold; baseline (speedup 1.0000x reference)
import jax
import jax.numpy as jnp
from jax.experimental import pallas as pl


def kernel(ids, vals, weight, bias):
    raise NotImplementedError("write your pallas kernel here")


# speedup vs baseline: 1.1179x; 1.1179x over previous
import functools

import jax
import jax.numpy as jnp
from jax import lax
from jax.experimental import pallas as pl
from jax.experimental.pallas import tpu as pltpu
from jax.experimental.pallas import tpu_sc as plsc

B = 16384
F = 100
NFEAT = 1000000

NC = 2
NS = 16
NW = NC * NS
ROWS_W = B // NW
CHUNK = 128
NCHUNK = ROWS_W // CHUNK
CW = CHUNK * F
NROW = CW // 128


def _build_sc_lr():
    mesh = plsc.VectorSubcoreMesh(core_axis_name="c", subcore_axis_name="s")

    @functools.partial(
        pl.kernel,
        mesh=mesh,
        compiler_params=pltpu.CompilerParams(needs_layout_passes=False),
        out_type=jax.ShapeDtypeStruct((B,), jnp.float32),
        scratch_types=[
            pltpu.VMEM((CW,), jnp.int32),
            pltpu.VMEM((CW,), jnp.float32),
            pltpu.VMEM((CW,), jnp.float32),
            pltpu.VMEM((CHUNK,), jnp.float32),
            pltpu.SemaphoreType.DMA,
        ],
    )
    def k(ids_hbm, vals_hbm, w_hbm, out_hbm, ids_v, vals_v, emb_v, acc_v, sem):
        wid = lax.axis_index("s") * NC + lax.axis_index("c")
        lane = lax.iota(jnp.int32, 16)
        for c in range(NCHUNK):
            slab = pl.multiple_of(wid * (NCHUNK * CW) + c * CW, 128)
            r0 = pl.multiple_of(wid * ROWS_W + c * CHUNK, 128)
            pltpu.sync_copy(ids_hbm.at[pl.ds(slab, CW)], ids_v)

            def fire(j, carry):
                off = pl.multiple_of(j * 128, 128)
                pltpu.async_copy(
                    w_hbm.at[ids_v.at[pl.ds(off, 128)]],
                    emb_v.at[pl.ds(off, 128)],
                    sem,
                )
                return carry

            lax.fori_loop(0, NROW, fire, 0)
            pltpu.sync_copy(vals_hbm.at[pl.ds(slab, CW)], vals_v)
            pltpu.make_async_copy(
                vals_hbm.at[pl.ds(slab, CW)], emb_v, sem
            ).wait()
            for j in range(CHUNK // 16):
                pj = lane * F + j * 16 * F

                def f_body(f, acc, pj=pj):
                    p = pj + f
                    e = plsc.load_gather(emb_v, [p])
                    v = plsc.load_gather(vals_v, [p])
                    return acc + e * v

                acc = lax.fori_loop(0, F, f_body, jnp.zeros((16,), jnp.float32))
                acc_v[pl.ds(j * 16, 16)] = acc
            pltpu.sync_copy(acc_v, out_hbm.at[pl.ds(r0, CHUNK)])

    return k


_SC_LR = _build_sc_lr()


def kernel(ids, vals, weight, bias):
    ids1 = ids.astype(jnp.int32).reshape(B * F)
    vals1 = vals.reshape(B * F)
    w1 = weight.reshape(NFEAT)
    y = _SC_LR(ids1, vals1, w1)
    return y + bias


# speedup vs baseline: 1.2239x; 1.0948x over previous
import functools

import jax
import jax.numpy as jnp
from jax import lax
from jax.experimental import pallas as pl
from jax.experimental.pallas import tpu as pltpu
from jax.experimental.pallas import tpu_sc as plsc

B = 16384
F = 100
NFEAT = 1000000

NC = 2
NS = 16
NW = NC * NS
ROWS_W = B // NW
CHUNK = 128
NCHUNK = ROWS_W // CHUNK
CW = CHUNK * F
NROW = CW // 128
NJ = CHUNK // 16


def _build_sc_lr():
    mesh = plsc.VectorSubcoreMesh(core_axis_name="c", subcore_axis_name="s")

    @functools.partial(
        pl.kernel,
        mesh=mesh,
        compiler_params=pltpu.CompilerParams(needs_layout_passes=False),
        out_type=jax.ShapeDtypeStruct((B,), jnp.float32),
        scratch_types=[
            pltpu.VMEM((CW,), jnp.int32),
            pltpu.VMEM((CW,), jnp.int32),
            pltpu.VMEM((CW,), jnp.float32),
            pltpu.VMEM((CW,), jnp.float32),
            pltpu.VMEM((CW,), jnp.float32),
            pltpu.VMEM((CW,), jnp.float32),
            pltpu.VMEM((CHUNK,), jnp.float32),
            pltpu.SemaphoreType.DMA,
            pltpu.SemaphoreType.DMA,
        ],
    )
    def k(ids_hbm, vals_hbm, w_hbm, out_hbm,
          ids_v0, ids_v1, vals_v0, vals_v1, emb_v0, emb_v1, acc_v,
          sem0, sem1):
        wid = lax.axis_index("s") * NC + lax.axis_index("c")
        lane = lax.iota(jnp.int32, 16)
        ids_b = (ids_v0, ids_v1)
        vals_b = (vals_v0, vals_v1)
        emb_b = (emb_v0, emb_v1)
        sem_b = (sem0, sem1)

        def slab_of(c):
            return pl.multiple_of(wid * (NCHUNK * CW) + c * CW, 128)

        def fire_chunk(c):
            q = c % 2
            slab = slab_of(c)
            pltpu.sync_copy(ids_hbm.at[pl.ds(slab, CW)], ids_b[q])

            def fire(j, carry):
                off = pl.multiple_of(j * 128, 128)
                pltpu.async_copy(
                    w_hbm.at[ids_b[q].at[pl.ds(off, 128)]],
                    emb_b[q].at[pl.ds(off, 128)],
                    sem_b[q],
                )
                return carry

            lax.fori_loop(0, NROW, fire, 0)
            pltpu.sync_copy(vals_hbm.at[pl.ds(slab, CW)], vals_b[q])

        fire_chunk(0)
        for c in range(NCHUNK):
            q = c % 2
            if c + 1 < NCHUNK:
                fire_chunk(c + 1)
            pltpu.make_async_copy(
                vals_hbm.at[pl.ds(slab_of(c), CW)], emb_b[q], sem_b[q]
            ).wait()

            pjs = tuple(lane * F + j * 16 * F for j in range(NJ))

            def f_body(f, accs, q=q, pjs=pjs):
                out = []
                for j in range(NJ):
                    p = pjs[j] + f
                    e = plsc.load_gather(emb_b[q], [p])
                    v = plsc.load_gather(vals_b[q], [p])
                    out.append(accs[j] + e * v)
                return tuple(out)

            accs = lax.fori_loop(
                0, F, f_body, (jnp.zeros((16,), jnp.float32),) * NJ
            )
            for j in range(NJ):
                acc_v[pl.ds(j * 16, 16)] = accs[j]
            r0 = pl.multiple_of(wid * ROWS_W + c * CHUNK, 128)
            pltpu.sync_copy(acc_v, out_hbm.at[pl.ds(r0, CHUNK)])

    return k


_SC_LR = _build_sc_lr()


def kernel(ids, vals, weight, bias):
    ids1 = ids.astype(jnp.int32).reshape(B * F)
    vals1 = vals.reshape(B * F)
    w1 = weight.reshape(NFEAT)
    y = _SC_LR(ids1, vals1, w1)
    return y + bias
